# baseline (speedup 1.0000x reference)
import jax
import jax.numpy as jnp
from jax import lax
from jax.experimental import pallas as pl
from jax.experimental.pallas import tpu as pltpu
from jax.experimental.pallas import tpu_sc as plsc

N_CLASSES = 1000000
D = 128
C1, C2, C3 = 10000, 60000, 190000
T2 = N_CLASSES - C3
P_ROWS = 14336
TBLK = 2000
N_TOK = 1024 * 200

NC, NS = 2, 16
NW = NC * NS
TOK_PER_W = N_TOK // NW
CHUNK = 128
N_CHUNKS = TOK_PER_W // CHUNK

PR_PER_W = P_ROWS // NS
PR_CHUNK = 128
N_PR_CHUNKS = PR_PER_W // PR_CHUNK


def _table_body(hemb, hw, te0, tl0, te1, tl1, out):
    pid = pl.program_id(0)

    @pl.when(pid < C1 // TBLK)
    def _():
        out[...] = jnp.dot(hemb[...], hw[...].T,
                           preferred_element_type=jnp.float32)

    @pl.when((pid >= C1 // TBLK) & (pid < C2 // TBLK))
    def _():
        out[...] = jnp.dot(te0[...], tl0[...],
                           preferred_element_type=jnp.float32)

    @pl.when(pid >= C2 // TBLK)
    def _():
        out[...] = jnp.dot(te1[...], tl1[...],
                           preferred_element_type=jnp.float32)


def _build_table(hemb, hw, te0, tl0, te1, tl1):
    b0, b1 = C1 // TBLK, C2 // TBLK
    nblk = C3 // TBLK
    return pl.pallas_call(
        _table_body,
        grid=(nblk,),
        in_specs=[
            pl.BlockSpec((TBLK, D), lambda i: (jnp.minimum(i, b0 - 1), 0)),
            pl.BlockSpec((D, D), lambda i: (0, 0)),
            pl.BlockSpec((TBLK, 32),
                         lambda i: (jnp.clip(i - b0, 0, (C2 - C1) // TBLK - 1), 0)),
            pl.BlockSpec((32, D), lambda i: (0, 0)),
            pl.BlockSpec((TBLK, 8),
                         lambda i: (jnp.clip(i - b1, 0, (C3 - C2) // TBLK - 1), 0)),
            pl.BlockSpec((8, D), lambda i: (0, 0)),
        ],
        out_specs=pl.BlockSpec((TBLK, D), lambda i: (i, 0)),
        out_shape=jax.ShapeDtypeStruct((C3, D), jnp.float32),
    )(hemb, hw, te0, tl0, te1, tl1)


def _gather_body(idx_hbm, x_hbm, lo_hbm, hi_hbm, tl2_hbm, g_out, p_out,
                 idx_v, tidx_v, pidx_v, lane_v, rows_v, rowsb_v,
                 e2lo_v, e2hi_v, l2_v, lo1d_v, hi1d_v, prow_v,
                 sem1, sem2):
    cidx = lax.axis_index("c")
    sidx = lax.axis_index("s")
    wid = sidx * NC + cidx
    base = wid * TOK_PER_W
    pltpu.sync_copy(tl2_hbm, l2_v)
    zero16 = lax.iota(jnp.int32, 16) * 0

    pbase = P_ROWS * cidx
    for k in range(N_PR_CHUNKS):
        pr0 = PR_PER_W * sidx + PR_CHUNK * k
        pltpu.sync_copy(lo_hbm.at[pl.ds(64 * pr0, 64 * PR_CHUNK)], lo1d_v)
        pltpu.sync_copy(hi_hbm.at[pl.ds(64 * pr0, 64 * PR_CHUNK)], hi1d_v)

        def pack(r, carry):
            for j in range(4):
                src = pl.ds(64 * r + 16 * j, 16)
                prow_v[r, pl.ds(16 * j, 16)] = lo1d_v[src]
                prow_v[r, pl.ds(64 + 16 * j, 16)] = hi1d_v[src]
            return carry

        lax.fori_loop(0, PR_CHUNK, pack, 0)
        pltpu.sync_copy(prow_v, p_out.at[pl.ds(pbase + pr0, PR_CHUNK)])

    plsc.subcore_barrier()

    def body(c, carry):
        off = base + c * CHUNK
        pltpu.sync_copy(idx_hbm.at[pl.ds(off, CHUNK)], idx_v)
        for j in range(CHUNK // 16):
            sl = pl.ds(j * 16, 16)
            v = idx_v[sl]
            u = jnp.minimum(jnp.maximum(v - C3, 0), T2 - 1)
            tidx_v[sl] = jnp.minimum(v, C3 - 1)
            pidx_v[sl] = pbase + (u >> 6)
            lane_v[sl] = u & 63
        cp1 = pltpu.async_copy(x_hbm.at[tidx_v], rows_v, sem1)
        cp2 = pltpu.async_copy(p_out.at[pidx_v], rowsb_v, sem2)
        cp2.wait()
        for j in range(CHUNK // 16):
            sl = pl.ds(j * 16, 16)
            toks = lax.iota(jnp.int32, 16) + (j * 16)
            c0 = lane_v[sl]
            e2lo_v[sl] = plsc.load_gather(rowsb_v, [toks, c0])
            e2hi_v[sl] = plsc.load_gather(rowsb_v, [toks, c0 + 64])
        cp1.wait()

        def fixup(t, carry2):
            tt = zero16 + t
            m = plsc.load_gather(idx_v, [tt]) >= C3
            a0 = plsc.load_gather(e2lo_v, [tt])
            a1 = plsc.load_gather(e2hi_v, [tt])
            for j in range(D // 16):
                dsl = pl.ds(j * 16, 16)
                val = a0 * l2_v[0, dsl] + a1 * l2_v[1, dsl]
                rows_v[t, dsl] = jnp.where(m, val, rows_v[t, dsl])
            return carry2

        lax.fori_loop(0, CHUNK, fixup, 0)
        pltpu.sync_copy(rows_v, g_out.at[pl.ds(off, CHUNK)])
        return carry

    lax.fori_loop(0, N_CHUNKS, body, 0)


def _gather_call():
    return pl.kernel(
        _gather_body,
        out_type=(jax.ShapeDtypeStruct((N_TOK, D), jnp.float32),
                  jax.ShapeDtypeStruct((NC * P_ROWS, D), jnp.float32)),
        mesh=plsc.VectorSubcoreMesh(core_axis_name="c", subcore_axis_name="s"),
        compiler_params=pltpu.CompilerParams(needs_layout_passes=False),
        scratch_types=[
            pltpu.VMEM((CHUNK,), jnp.int32),
            pltpu.VMEM((CHUNK,), jnp.int32),
            pltpu.VMEM((CHUNK,), jnp.int32),
            pltpu.VMEM((CHUNK,), jnp.int32),
            pltpu.VMEM((CHUNK, D), jnp.float32),
            pltpu.VMEM((CHUNK, D), jnp.float32),
            pltpu.VMEM((CHUNK,), jnp.float32),
            pltpu.VMEM((CHUNK,), jnp.float32),
            pltpu.VMEM((2, D), jnp.float32),
            pltpu.VMEM((64 * PR_CHUNK,), jnp.float32),
            pltpu.VMEM((64 * PR_CHUNK,), jnp.float32),
            pltpu.VMEM((PR_CHUNK, D), jnp.float32),
            pltpu.SemaphoreType.DMA,
            pltpu.SemaphoreType.DMA,
        ],
    )


def kernel(input, head_emb, head_w, tail_emb_0, tail_lin_0,
           tail_emb_1, tail_lin_1, tail_emb_2, tail_lin_2):
    orig_shape = input.shape
    flat = input.reshape(-1)

    pad = P_ROWS * 64 - T2
    lo = jnp.pad(tail_emb_2[:, 0], (0, pad))
    hi = jnp.pad(tail_emb_2[:, 1], (0, pad))

    table = _build_table(head_emb, head_w, tail_emb_0, tail_lin_0,
                         tail_emb_1, tail_lin_1)

    out, _ = _gather_call()(flat, table, lo, hi, tail_lin_2)
    return out.reshape(orig_shape + (D,))


# speedup vs baseline: 10.0544x; 10.0544x over previous
import jax
import jax.numpy as jnp
from jax import lax
from jax.experimental import pallas as pl
from jax.experimental.pallas import tpu as pltpu
from jax.experimental.pallas import tpu_sc as plsc

N_CLASSES = 1000000
D = 128
C1, C2, C3 = 10000, 60000, 190000
T2 = N_CLASSES - C3
T2_ROWS = 14000
X_ROWS = C3 + T2_ROWS
TBLK = 2000
N_TOK = 1024 * 200

NC, NS = 2, 16
NW = NC * NS
TOK_PER_W = N_TOK // NW
CHUNK = 128
N_CHUNKS = TOK_PER_W // CHUNK

MB = 2048
N_MB = N_TOK // MB


def _table_body(hemb, hw, te0, tl0, te1, tl1, t2p, out):
    pid = pl.program_id(0)

    @pl.when(pid < C1 // TBLK)
    def _():
        out[...] = jnp.dot(hemb[...], hw[...].T,
                           preferred_element_type=jnp.float32)

    @pl.when((pid >= C1 // TBLK) & (pid < C2 // TBLK))
    def _():
        out[...] = jnp.dot(te0[...], tl0[...],
                           preferred_element_type=jnp.float32)

    @pl.when((pid >= C2 // TBLK) & (pid < C3 // TBLK))
    def _():
        out[...] = jnp.dot(te1[...], tl1[...],
                           preferred_element_type=jnp.float32)

    @pl.when(pid >= C3 // TBLK)
    def _():
        out[...] = t2p[...]


def _build_table(hemb, hw, te0, tl0, te1, tl1, t2p):
    b0, b1, b2 = C1 // TBLK, C2 // TBLK, C3 // TBLK
    nblk = X_ROWS // TBLK
    return pl.pallas_call(
        _table_body,
        grid=(nblk,),
        in_specs=[
            pl.BlockSpec((TBLK, D), lambda i: (jnp.minimum(i, b0 - 1), 0)),
            pl.BlockSpec((D, D), lambda i: (0, 0)),
            pl.BlockSpec((TBLK, 32),
                         lambda i: (jnp.clip(i - b0, 0, (C2 - C1) // TBLK - 1), 0)),
            pl.BlockSpec((32, D), lambda i: (0, 0)),
            pl.BlockSpec((TBLK, 8),
                         lambda i: (jnp.clip(i - b1, 0, (C3 - C2) // TBLK - 1), 0)),
            pl.BlockSpec((8, D), lambda i: (0, 0)),
            pl.BlockSpec((TBLK, D),
                         lambda i: (jnp.clip(i - b2, 0, T2_ROWS // TBLK - 1), 0)),
        ],
        out_specs=pl.BlockSpec((TBLK, D), lambda i: (i, 0)),
        out_shape=jax.ShapeDtypeStruct((X_ROWS, D), jnp.float32),
    )(hemb, hw, te0, tl0, te1, tl1, t2p)


def _gather_body(idx_hbm, x_hbm, g_out, e2lo_out, e2hi_out,
                 idx_v, tidx_v, lane_v, rows_v, e2lo_v, e2hi_v, sem1):
    wid = lax.axis_index("s") * NC + lax.axis_index("c")
    base = wid * TOK_PER_W

    def body(c, carry):
        off = base + c * CHUNK
        pltpu.sync_copy(idx_hbm.at[pl.ds(off, CHUNK)], idx_v)
        for j in range(CHUNK // 16):
            sl = pl.ds(j * 16, 16)
            v = idx_v[sl]
            u = jnp.minimum(jnp.maximum(v - C3, 0), T2 - 1)
            tidx_v[sl] = jnp.where(v < C3, v, C3 + (u >> 6))
            lane_v[sl] = u & 63
        cp1 = pltpu.async_copy(x_hbm.at[tidx_v], rows_v, sem1)
        cp1.wait()
        for j in range(CHUNK // 16):
            sl = pl.ds(j * 16, 16)
            toks = lax.iota(jnp.int32, 16) + (j * 16)
            c0 = lane_v[sl]
            e2lo_v[sl] = plsc.load_gather(rows_v, [toks, c0])
            e2hi_v[sl] = plsc.load_gather(rows_v, [toks, c0 + 64])
        pltpu.sync_copy(rows_v, g_out.at[pl.ds(off, CHUNK)])
        pltpu.sync_copy(e2lo_v, e2lo_out.at[pl.ds(off, CHUNK)])
        pltpu.sync_copy(e2hi_v, e2hi_out.at[pl.ds(off, CHUNK)])
        return carry

    lax.fori_loop(0, N_CHUNKS, body, 0)


def _gather_call():
    return pl.kernel(
        _gather_body,
        out_type=(jax.ShapeDtypeStruct((N_TOK, D), jnp.float32),
                  jax.ShapeDtypeStruct((N_TOK,), jnp.float32),
                  jax.ShapeDtypeStruct((N_TOK,), jnp.float32)),
        mesh=plsc.VectorSubcoreMesh(core_axis_name="c", subcore_axis_name="s"),
        compiler_params=pltpu.CompilerParams(needs_layout_passes=False),
        scratch_types=[
            pltpu.VMEM((CHUNK,), jnp.int32),
            pltpu.VMEM((CHUNK,), jnp.int32),
            pltpu.VMEM((CHUNK,), jnp.int32),
            pltpu.VMEM((CHUNK, D), jnp.float32),
            pltpu.VMEM((CHUNK,), jnp.float32),
            pltpu.VMEM((CHUNK,), jnp.float32),
            pltpu.SemaphoreType.DMA,
        ],
    )


def _merge_body(idx_ref, g_ref, e2lo_ref, e2hi_ref, tl2_ref, out_ref):
    mask = idx_ref[...] < C3
    dense = (e2lo_ref[...] * tl2_ref[0:1, :]
             + e2hi_ref[...] * tl2_ref[1:2, :])
    out_ref[...] = jnp.where(mask, g_ref[...], dense)


def _merge(idx_col, g, e2lo, e2hi, tl2):
    return pl.pallas_call(
        _merge_body,
        grid=(N_MB,),
        in_specs=[
            pl.BlockSpec((MB, 1), lambda i: (i, 0)),
            pl.BlockSpec((MB, D), lambda i: (i, 0)),
            pl.BlockSpec((MB, 1), lambda i: (i, 0)),
            pl.BlockSpec((MB, 1), lambda i: (i, 0)),
            pl.BlockSpec((2, D), lambda i: (0, 0)),
        ],
        out_specs=pl.BlockSpec((MB, D), lambda i: (i, 0)),
        out_shape=jax.ShapeDtypeStruct((N_TOK, D), jnp.float32),
    )(idx_col, g, e2lo, e2hi, tl2)


def kernel(input, head_emb, head_w, tail_emb_0, tail_lin_0,
           tail_emb_1, tail_lin_1, tail_emb_2, tail_lin_2):
    orig_shape = input.shape
    flat = input.reshape(-1)

    pad = T2_ROWS * 64 - T2
    lo2 = jnp.pad(tail_emb_2[:, 0], (0, pad)).reshape(T2_ROWS, 64)
    hi2 = jnp.pad(tail_emb_2[:, 1], (0, pad)).reshape(T2_ROWS, 64)
    t2p = jnp.concatenate([lo2, hi2], axis=1)

    table = _build_table(head_emb, head_w, tail_emb_0, tail_lin_0,
                         tail_emb_1, tail_lin_1, t2p)

    gathered, e2lo, e2hi = _gather_call()(flat, table)

    idx_col = flat.reshape(N_TOK, 1)
    out = _merge(idx_col, gathered, e2lo.reshape(N_TOK, 1),
                 e2hi.reshape(N_TOK, 1), tail_lin_2)
    return out.reshape(orig_shape + (D,))


# speedup vs baseline: 10.6897x; 1.0632x over previous
import jax
import jax.numpy as jnp
from jax import lax
from jax.experimental import pallas as pl
from jax.experimental.pallas import tpu as pltpu
from jax.experimental.pallas import tpu_sc as plsc

N_CLASSES = 1000000
D = 128
C1, C2, C3 = 10000, 60000, 190000
T2 = N_CLASSES - C3
T2_ROWS = 14000
X_ROWS = C3 + T2_ROWS
TBLK = 2000
N_TOK = 1024 * 200

NC, NS = 2, 16
NW = NC * NS
TOK_PER_W = N_TOK // NW
CHUNK = 640
SUB = 128
N_CHUNKS = TOK_PER_W // CHUNK

MB = 2048
N_MB = N_TOK // MB


def _table_body(hemb, hw, te0, tl0, te1, tl1, t2p, out):
    pid = pl.program_id(0)

    @pl.when(pid < C1 // TBLK)
    def _():
        out[...] = jnp.dot(hemb[...], hw[...].T,
                           preferred_element_type=jnp.float32)

    @pl.when((pid >= C1 // TBLK) & (pid < C2 // TBLK))
    def _():
        out[...] = jnp.dot(te0[...], tl0[...],
                           preferred_element_type=jnp.float32)

    @pl.when((pid >= C2 // TBLK) & (pid < C3 // TBLK))
    def _():
        out[...] = jnp.dot(te1[...], tl1[...],
                           preferred_element_type=jnp.float32)

    @pl.when(pid >= C3 // TBLK)
    def _():
        out[...] = t2p[...]


def _build_table(hemb, hw, te0, tl0, te1, tl1, t2p):
    b0, b1, b2 = C1 // TBLK, C2 // TBLK, C3 // TBLK
    nblk = X_ROWS // TBLK
    return pl.pallas_call(
        _table_body,
        grid=(nblk,),
        in_specs=[
            pl.BlockSpec((TBLK, D), lambda i: (jnp.minimum(i, b0 - 1), 0)),
            pl.BlockSpec((D, D), lambda i: (0, 0)),
            pl.BlockSpec((TBLK, 32),
                         lambda i: (jnp.clip(i - b0, 0, (C2 - C1) // TBLK - 1), 0)),
            pl.BlockSpec((32, D), lambda i: (0, 0)),
            pl.BlockSpec((TBLK, 8),
                         lambda i: (jnp.clip(i - b1, 0, (C3 - C2) // TBLK - 1), 0)),
            pl.BlockSpec((8, D), lambda i: (0, 0)),
            pl.BlockSpec((TBLK, D),
                         lambda i: (jnp.clip(i - b2, 0, T2_ROWS // TBLK - 1), 0)),
        ],
        out_specs=pl.BlockSpec((TBLK, D), lambda i: (i, 0)),
        out_shape=jax.ShapeDtypeStruct((X_ROWS, D), jnp.float32),
    )(hemb, hw, te0, tl0, te1, tl1, t2p)


def _gather_body(idx_hbm, x_hbm, g_out, e2lo_out, e2hi_out,
                 idx_v, tidx_v, lane_v, rows_v, e2lo_v, e2hi_v, sem1):
    wid = lax.axis_index("s") * NC + lax.axis_index("c")
    base = wid * TOK_PER_W

    def body(c, carry):
        off = base + c * CHUNK
        pltpu.sync_copy(idx_hbm.at[pl.ds(off, CHUNK)], idx_v)
        for j in range(CHUNK // 16):
            sl = pl.ds(j * 16, 16)
            v = idx_v[sl]
            u = jnp.minimum(jnp.maximum(v - C3, 0), T2 - 1)
            tidx_v[sl] = jnp.where(v < C3, v, C3 + (u >> 6))
            lane_v[sl] = u & 63
        cps = [pltpu.async_copy(x_hbm.at[tidx_v.at[pl.ds(k * SUB, SUB)]],
                                rows_v.at[pl.ds(k * SUB, SUB), :], sem1)
               for k in range(CHUNK // SUB)]
        for cp in cps:
            cp.wait()
        for j in range(CHUNK // 16):
            sl = pl.ds(j * 16, 16)
            toks = lax.iota(jnp.int32, 16) + (j * 16)
            c0 = lane_v[sl]
            e2lo_v[sl] = plsc.load_gather(rows_v, [toks, c0])
            e2hi_v[sl] = plsc.load_gather(rows_v, [toks, c0 + 64])
        pltpu.sync_copy(rows_v, g_out.at[pl.ds(off, CHUNK)])
        pltpu.sync_copy(e2lo_v, e2lo_out.at[pl.ds(off, CHUNK)])
        pltpu.sync_copy(e2hi_v, e2hi_out.at[pl.ds(off, CHUNK)])
        return carry

    lax.fori_loop(0, N_CHUNKS, body, 0)


def _gather_call():
    return pl.kernel(
        _gather_body,
        out_type=(jax.ShapeDtypeStruct((N_TOK, D), jnp.float32),
                  jax.ShapeDtypeStruct((N_TOK,), jnp.float32),
                  jax.ShapeDtypeStruct((N_TOK,), jnp.float32)),
        mesh=plsc.VectorSubcoreMesh(core_axis_name="c", subcore_axis_name="s"),
        compiler_params=pltpu.CompilerParams(needs_layout_passes=False),
        scratch_types=[
            pltpu.VMEM((CHUNK,), jnp.int32),
            pltpu.VMEM((CHUNK,), jnp.int32),
            pltpu.VMEM((CHUNK,), jnp.int32),
            pltpu.VMEM((CHUNK, D), jnp.float32),
            pltpu.VMEM((CHUNK,), jnp.float32),
            pltpu.VMEM((CHUNK,), jnp.float32),
            pltpu.SemaphoreType.DMA,
        ],
    )


def _merge_body(idx_ref, g_ref, e2lo_ref, e2hi_ref, tl2_ref, out_ref):
    mask = idx_ref[...] < C3
    dense = (e2lo_ref[...] * tl2_ref[0:1, :]
             + e2hi_ref[...] * tl2_ref[1:2, :])
    out_ref[...] = jnp.where(mask, g_ref[...], dense)


def _merge(idx_col, g, e2lo, e2hi, tl2):
    return pl.pallas_call(
        _merge_body,
        grid=(N_MB,),
        in_specs=[
            pl.BlockSpec((MB, 1), lambda i: (i, 0)),
            pl.BlockSpec((MB, D), lambda i: (i, 0)),
            pl.BlockSpec((MB, 1), lambda i: (i, 0)),
            pl.BlockSpec((MB, 1), lambda i: (i, 0)),
            pl.BlockSpec((2, D), lambda i: (0, 0)),
        ],
        out_specs=pl.BlockSpec((MB, D), lambda i: (i, 0)),
        out_shape=jax.ShapeDtypeStruct((N_TOK, D), jnp.float32),
    )(idx_col, g, e2lo, e2hi, tl2)


def kernel(input, head_emb, head_w, tail_emb_0, tail_lin_0,
           tail_emb_1, tail_lin_1, tail_emb_2, tail_lin_2):
    orig_shape = input.shape
    flat = input.reshape(-1)

    pad = T2_ROWS * 64 - T2
    lo2 = jnp.pad(tail_emb_2[:, 0], (0, pad)).reshape(T2_ROWS, 64)
    hi2 = jnp.pad(tail_emb_2[:, 1], (0, pad)).reshape(T2_ROWS, 64)
    t2p = jnp.concatenate([lo2, hi2], axis=1)

    table = _build_table(head_emb, head_w, tail_emb_0, tail_lin_0,
                         tail_emb_1, tail_lin_1, t2p)

    gathered, e2lo, e2hi = _gather_call()(flat, table)

    idx_col = flat.reshape(N_TOK, 1)
    out = _merge(idx_col, gathered, e2lo.reshape(N_TOK, 1),
                 e2hi.reshape(N_TOK, 1), tail_lin_2)
    return out.reshape(orig_shape + (D,))


# speedup vs baseline: 10.7474x; 1.0054x over previous
import jax
import jax.numpy as jnp
from jax import lax
from jax.experimental import pallas as pl
from jax.experimental.pallas import tpu as pltpu
from jax.experimental.pallas import tpu_sc as plsc

N_CLASSES = 1000000
D = 128
C1, C2, C3 = 10000, 60000, 190000
T2 = N_CLASSES - C3
T2_ROWS = 14000
X_ROWS = C3 + T2_ROWS
TBLK = 2000
N_TOK = 1024 * 200

NC, NS = 2, 16
NW = NC * NS
TOK_PER_W = N_TOK // NW
CHUNK = 640
SUB = 128
N_CHUNKS = TOK_PER_W // CHUNK

MB = 2048
N_MB = N_TOK // MB


def _table_body(hemb, hw, te0, tl0, te1, tl1, t2p, out):
    pid = pl.program_id(0)

    @pl.when(pid < C1 // TBLK)
    def _():
        out[...] = jnp.dot(hemb[...], hw[...].T,
                           preferred_element_type=jnp.float32)

    @pl.when((pid >= C1 // TBLK) & (pid < C2 // TBLK))
    def _():
        out[...] = jnp.dot(te0[...], tl0[...],
                           preferred_element_type=jnp.float32)

    @pl.when((pid >= C2 // TBLK) & (pid < C3 // TBLK))
    def _():
        out[...] = jnp.dot(te1[...], tl1[...],
                           preferred_element_type=jnp.float32)

    @pl.when(pid >= C3 // TBLK)
    def _():
        out[...] = t2p[...]


def _build_table(hemb, hw, te0, tl0, te1, tl1, t2p):
    b0, b1, b2 = C1 // TBLK, C2 // TBLK, C3 // TBLK
    nblk = X_ROWS // TBLK
    return pl.pallas_call(
        _table_body,
        grid=(nblk,),
        in_specs=[
            pl.BlockSpec((TBLK, D), lambda i: (jnp.minimum(i, b0 - 1), 0)),
            pl.BlockSpec((D, D), lambda i: (0, 0)),
            pl.BlockSpec((TBLK, 32),
                         lambda i: (jnp.clip(i - b0, 0, (C2 - C1) // TBLK - 1), 0)),
            pl.BlockSpec((32, D), lambda i: (0, 0)),
            pl.BlockSpec((TBLK, 8),
                         lambda i: (jnp.clip(i - b1, 0, (C3 - C2) // TBLK - 1), 0)),
            pl.BlockSpec((8, D), lambda i: (0, 0)),
            pl.BlockSpec((TBLK, D),
                         lambda i: (jnp.clip(i - b2, 0, T2_ROWS // TBLK - 1), 0)),
        ],
        out_specs=pl.BlockSpec((TBLK, D), lambda i: (i, 0)),
        out_shape=jax.ShapeDtypeStruct((X_ROWS, D), jnp.float32),
    )(hemb, hw, te0, tl0, te1, tl1, t2p)


def _gather_body(idx_hbm, x_hbm, g_out, e2lo_out, e2hi_out,
                 idx_v, tidx_v, lane_v, rows_v, e2lo_v, e2hi_v, sem1, sem2):
    wid = lax.axis_index("s") * NC + lax.axis_index("c")
    base = wid * TOK_PER_W

    def body(c, carry):
        off = base + c * CHUNK
        pltpu.sync_copy(idx_hbm.at[pl.ds(off, CHUNK)], idx_v)
        for j in range(CHUNK // 16):
            sl = pl.ds(j * 16, 16)
            v = idx_v[sl]
            u = jnp.minimum(jnp.maximum(v - C3, 0), T2 - 1)
            tidx_v[sl] = jnp.where(v < C3, v, C3 + (u >> 6))
            lane_v[sl] = u & 63
        cps = [pltpu.async_copy(x_hbm.at[tidx_v.at[pl.ds(k * SUB, SUB)]],
                                rows_v.at[pl.ds(k * SUB, SUB), :], sem1)
               for k in range(CHUNK // SUB)]
        for cp in cps:
            cp.wait()
        wbg = pltpu.async_copy(rows_v, g_out.at[pl.ds(off, CHUNK)], sem2)
        for j in range(CHUNK // 16):
            sl = pl.ds(j * 16, 16)
            toks = lax.iota(jnp.int32, 16) + (j * 16)
            c0 = lane_v[sl]
            e2lo_v[sl] = plsc.load_gather(rows_v, [toks, c0])
            e2hi_v[sl] = plsc.load_gather(rows_v, [toks, c0 + 64])
        pltpu.sync_copy(e2lo_v, e2lo_out.at[pl.ds(off, CHUNK)])
        pltpu.sync_copy(e2hi_v, e2hi_out.at[pl.ds(off, CHUNK)])
        wbg.wait()
        return carry

    lax.fori_loop(0, N_CHUNKS, body, 0)


def _gather_call():
    return pl.kernel(
        _gather_body,
        out_type=(jax.ShapeDtypeStruct((N_TOK, D), jnp.float32),
                  jax.ShapeDtypeStruct((N_TOK,), jnp.float32),
                  jax.ShapeDtypeStruct((N_TOK,), jnp.float32)),
        mesh=plsc.VectorSubcoreMesh(core_axis_name="c", subcore_axis_name="s"),
        compiler_params=pltpu.CompilerParams(needs_layout_passes=False),
        scratch_types=[
            pltpu.VMEM((CHUNK,), jnp.int32),
            pltpu.VMEM((CHUNK,), jnp.int32),
            pltpu.VMEM((CHUNK,), jnp.int32),
            pltpu.VMEM((CHUNK, D), jnp.float32),
            pltpu.VMEM((CHUNK,), jnp.float32),
            pltpu.VMEM((CHUNK,), jnp.float32),
            pltpu.SemaphoreType.DMA,
            pltpu.SemaphoreType.DMA,
        ],
    )


def _merge_body(idx_ref, g_ref, e2lo_ref, e2hi_ref, tl2_ref, out_ref):
    mask = idx_ref[...] < C3
    dense = (e2lo_ref[...] * tl2_ref[0:1, :]
             + e2hi_ref[...] * tl2_ref[1:2, :])
    out_ref[...] = jnp.where(mask, g_ref[...], dense)


def _merge(idx_col, g, e2lo, e2hi, tl2):
    return pl.pallas_call(
        _merge_body,
        grid=(N_MB,),
        in_specs=[
            pl.BlockSpec((MB, 1), lambda i: (i, 0)),
            pl.BlockSpec((MB, D), lambda i: (i, 0)),
            pl.BlockSpec((MB, 1), lambda i: (i, 0)),
            pl.BlockSpec((MB, 1), lambda i: (i, 0)),
            pl.BlockSpec((2, D), lambda i: (0, 0)),
        ],
        out_specs=pl.BlockSpec((MB, D), lambda i: (i, 0)),
        out_shape=jax.ShapeDtypeStruct((N_TOK, D), jnp.float32),
    )(idx_col, g, e2lo, e2hi, tl2)


def kernel(input, head_emb, head_w, tail_emb_0, tail_lin_0,
           tail_emb_1, tail_lin_1, tail_emb_2, tail_lin_2):
    orig_shape = input.shape
    flat = input.reshape(-1)

    pad = T2_ROWS * 64 - T2
    lo2 = jnp.pad(tail_emb_2[:, 0], (0, pad)).reshape(T2_ROWS, 64)
    hi2 = jnp.pad(tail_emb_2[:, 1], (0, pad)).reshape(T2_ROWS, 64)
    t2p = jnp.concatenate([lo2, hi2], axis=1)

    table = _build_table(head_emb, head_w, tail_emb_0, tail_lin_0,
                         tail_emb_1, tail_lin_1, t2p)

    gathered, e2lo, e2hi = _gather_call()(flat, table)

    idx_col = flat.reshape(N_TOK, 1)
    out = _merge(idx_col, gathered, e2lo.reshape(N_TOK, 1),
                 e2hi.reshape(N_TOK, 1), tail_lin_2)
    return out.reshape(orig_shape + (D,))
